# baseline (speedup 1.0000x reference)
import functools
import math

import jax
import jax.numpy as jnp
from jax import lax
from jax.experimental import pallas as pl
from jax.experimental.pallas import tpu as pltpu
from jax.experimental.pallas import tpu_sc as plsc

SPAT_DIM = 192
TIME_DIM = 192
MAX_N_TIMES = int(600.0 * 4.0)

LANES = 16
CHUNK = 128
TSTAGE = 32


def _time_table(n_times, n_dim, max_n_times):
    position = jnp.arange(n_times, dtype=jnp.float32)[:, None]
    div = jnp.exp(
        jnp.arange(0, n_dim, 2, dtype=jnp.float32) * (-math.log(max_n_times) / n_dim)
    )
    ang = position * div
    return jnp.stack([jnp.sin(ang), jnp.cos(ang)], axis=-1).reshape(n_times, n_dim)


def _sc_body(idx_hbm, emb_hbm, tt_hbm, out_hbm,
             idx_v, spat_v, tt_stage0, tt_stage1, buf0, buf1,
             sem_g, sem_t0, sem_t1, sem0, sem1,
             *, n_chans, n_times):
    core = lax.axis_index("c")
    sub = lax.axis_index("s")
    w = sub * 2 + core

    pltpu.sync_copy(idx_hbm.at[w], idx_v)
    gather = pltpu.make_async_copy(emb_hbm.at[idx_v], spat_v, sem_g)
    gather.start()

    n_chunks = n_times // CHUNK
    kspat = SPAT_DIM // LANES
    ktime = TIME_DIM // LANES
    bufs = [buf0, buf1]
    sems = [sem0, sem1]
    stages = [tt_stage0, tt_stage1]
    tsems = [sem_t0, sem_t1]

    def compose_and_fire(c, q, prev_cp):
        buf, sem = bufs[q], sems[q]
        dst = out_hbm.at[w, pl.ds(c * n_times + q * CHUNK, CHUNK), :]
        cp = pltpu.make_async_copy(buf, dst, sem)
        if prev_cp is not None:
            prev_cp.wait()
        spat_regs = [spat_v[c, pl.ds(k * LANES, LANES)] for k in range(kspat)]

        def row_body(r, _, buf=buf, spat_regs=spat_regs):
            for k in range(kspat):
                buf[r, pl.ds(k * LANES, LANES)] = spat_regs[k]
            return 0

        lax.fori_loop(0, CHUNK, row_body, 0)
        cp.start()
        return cp

    prev = [None, None]
    slabs = CHUNK // TSTAGE
    n_stages = n_chunks * slabs
    scopies = []
    for t in range(n_stages):
        scopies.append(pltpu.make_async_copy(
            tt_hbm.at[pl.ds(t * TSTAGE, TSTAGE), :], stages[t & 1],
            tsems[t & 1]))
    scopies[0].start()
    for t in range(n_stages):
        if t + 1 < n_stages:
            scopies[t + 1].start()
        scopies[t].wait()
        q, s = divmod(t, slabs)

        def tt_body(r, _, buf=bufs[q], stage=stages[t & 1], s=s):
            for k in range(ktime):
                buf[s * TSTAGE + r, pl.ds(SPAT_DIM + k * LANES, LANES)] = (
                    stage[r, pl.ds(k * LANES, LANES)])
            return 0

        lax.fori_loop(0, TSTAGE, tt_body, 0)
        if t == slabs - 1:
            gather.wait()
            prev[0] = compose_and_fire(0, 0, None)
    prev[1] = compose_and_fire(0, 1, None)

    for c in range(1, n_chans):
        for q in range(n_chunks):
            prev[q] = compose_and_fire(c, q, prev[q])
    prev[0].wait()
    prev[1].wait()


def kernel(local_features, ch_idxs, emb_table):
    B, n_chans_times, emb_dim = local_features.shape
    n_chans = ch_idxs.shape[1]
    n_times = n_chans_times // n_chans

    tt = _time_table(n_times, TIME_DIM, MAX_N_TIMES).astype(local_features.dtype)
    emb_wide = 256
    emb_padded = jnp.pad(emb_table, ((0, 0), (0, emb_wide - SPAT_DIM)))
    tt_padded = jnp.pad(tt, ((0, 0), (0, emb_wide - TIME_DIM)))
    idx_wide = 24
    idx_padded = jnp.pad(ch_idxs, ((0, 0), (0, idx_wide - n_chans)))

    body = functools.partial(_sc_body, n_chans=n_chans, n_times=n_times)
    run = pl.kernel(
        body,
        out_type=jax.ShapeDtypeStruct((B, n_chans_times, emb_dim),
                                      local_features.dtype),
        mesh=plsc.VectorSubcoreMesh(core_axis_name="c", subcore_axis_name="s"),
        scratch_types=[
            pltpu.VMEM((idx_wide,), jnp.int32),
            pltpu.VMEM((idx_wide, emb_wide), jnp.float32),
            pltpu.VMEM((TSTAGE, emb_wide), jnp.float32),
            pltpu.VMEM((TSTAGE, emb_wide), jnp.float32),
            pltpu.VMEM((CHUNK, emb_dim), jnp.float32),
            pltpu.VMEM((CHUNK, emb_dim), jnp.float32),
            pltpu.SemaphoreType.DMA,
            pltpu.SemaphoreType.DMA,
            pltpu.SemaphoreType.DMA,
            pltpu.SemaphoreType.DMA,
            pltpu.SemaphoreType.DMA,
        ],
    )
    return run(idx_padded, emb_padded, tt_padded)


# speedup vs baseline: 1.0077x; 1.0077x over previous
import functools
import math

import jax
import jax.numpy as jnp
from jax import lax
from jax.experimental import pallas as pl
from jax.experimental.pallas import tpu as pltpu
from jax.experimental.pallas import tpu_sc as plsc

SPAT_DIM = 192
TIME_DIM = 192
MAX_N_TIMES = int(600.0 * 4.0)

LANES = 16
CHUNK = 128
TSTAGE = 32


def _time_table(n_times, n_dim, max_n_times):
    position = jnp.arange(n_times, dtype=jnp.float32)[:, None]
    div = jnp.exp(
        jnp.arange(0, n_dim, 2, dtype=jnp.float32) * (-math.log(max_n_times) / n_dim)
    )
    ang = position * div
    return jnp.stack([jnp.sin(ang), jnp.cos(ang)], axis=-1).reshape(n_times, n_dim)


def _sc_body(idx_hbm, emb_hbm, tt_hbm, out_hbm,
             idx_v, spat_v, tt_stage0, tt_stage1, buf0, buf1,
             sem_g, sem_t0, sem_t1, sem0, sem1,
             *, n_chans, n_times):
    core = lax.axis_index("c")
    sub = lax.axis_index("s")
    w = sub * 2 + core

    pltpu.sync_copy(idx_hbm.at[w], idx_v)
    gather = pltpu.make_async_copy(emb_hbm.at[idx_v], spat_v, sem_g)
    gather.start()

    n_chunks = n_times // CHUNK
    kspat = SPAT_DIM // LANES
    ktime = TIME_DIM // LANES
    bufs = [buf0, buf1]
    sems = [sem0, sem1]
    stages = [tt_stage0, tt_stage1]
    tsems = [sem_t0, sem_t1]

    def compose_and_fire(c, q, prev_cp):
        buf, sem = bufs[q], sems[q]
        dst = out_hbm.at[w, pl.ds(c * n_times + q * CHUNK, CHUNK), :]
        cp = pltpu.make_async_copy(buf, dst, sem)
        if prev_cp is not None:
            prev_cp.wait()
        spat_regs = [spat_v[c, pl.ds(k * LANES, LANES)] for k in range(kspat)]

        def row_body(r, _, buf=buf, spat_regs=spat_regs):
            for k in range(kspat):
                buf[r, pl.ds(k * LANES, LANES)] = spat_regs[k]
            return 0

        lax.fori_loop(0, CHUNK, row_body, 0)
        cp.start()
        return cp

    prev = [None, None]
    slabs = CHUNK // TSTAGE
    n_stages = n_chunks * slabs
    scopies = []
    for t in range(n_stages):
        scopies.append(pltpu.make_async_copy(
            tt_hbm.at[pl.ds(t * TSTAGE, TSTAGE), :], stages[t & 1],
            tsems[t & 1]))
    scopies[0].start()
    for t in range(n_stages):
        if t + 1 < n_stages:
            scopies[t + 1].start()
        scopies[t].wait()
        q, s = divmod(t, slabs)

        def tt_body(r, _, buf=bufs[q], stage=stages[t & 1], s=s):
            for k in range(ktime):
                buf[s * TSTAGE + r, pl.ds(SPAT_DIM + k * LANES, LANES)] = (
                    stage[r, pl.ds(k * LANES, LANES)])
            return 0

        lax.fori_loop(0, TSTAGE, tt_body, 0)
    gather.wait()

    for c in range(n_chans):
        for q in range(n_chunks):
            prev[q] = compose_and_fire(c, q, prev[q])
    prev[0].wait()
    prev[1].wait()


def kernel(local_features, ch_idxs, emb_table):
    B, n_chans_times, emb_dim = local_features.shape
    n_chans = ch_idxs.shape[1]
    n_times = n_chans_times // n_chans

    tt = _time_table(n_times, TIME_DIM, MAX_N_TIMES).astype(local_features.dtype)
    emb_wide = 256
    emb_padded = jnp.pad(emb_table, ((0, 0), (0, emb_wide - SPAT_DIM)))
    tt_padded = jnp.pad(tt, ((0, 0), (0, emb_wide - TIME_DIM)))
    idx_wide = 24
    idx_padded = jnp.pad(ch_idxs, ((0, 0), (0, idx_wide - n_chans)))

    body = functools.partial(_sc_body, n_chans=n_chans, n_times=n_times)
    run = pl.kernel(
        body,
        out_type=jax.ShapeDtypeStruct((B, n_chans_times, emb_dim),
                                      local_features.dtype),
        mesh=plsc.VectorSubcoreMesh(core_axis_name="c", subcore_axis_name="s"),
        scratch_types=[
            pltpu.VMEM((idx_wide,), jnp.int32),
            pltpu.VMEM((idx_wide, emb_wide), jnp.float32),
            pltpu.VMEM((TSTAGE, emb_wide), jnp.float32),
            pltpu.VMEM((TSTAGE, emb_wide), jnp.float32),
            pltpu.VMEM((CHUNK, emb_dim), jnp.float32),
            pltpu.VMEM((CHUNK, emb_dim), jnp.float32),
            pltpu.SemaphoreType.DMA,
            pltpu.SemaphoreType.DMA,
            pltpu.SemaphoreType.DMA,
            pltpu.SemaphoreType.DMA,
            pltpu.SemaphoreType.DMA,
        ],
    )
    return run(idx_padded, emb_padded, tt_padded)


# speedup vs baseline: 1.0183x; 1.0105x over previous
import functools
import math

import jax
import jax.numpy as jnp
from jax import lax
from jax.experimental import pallas as pl
from jax.experimental.pallas import tpu as pltpu
from jax.experimental.pallas import tpu_sc as plsc

SPAT_DIM = 192
TIME_DIM = 192
MAX_N_TIMES = int(600.0 * 4.0)

LANES = 16
CHUNK = 128
TSTAGE = 32


def _time_table(n_times, n_dim, max_n_times):
    position = jnp.arange(n_times, dtype=jnp.float32)[:, None]
    div = jnp.exp(
        jnp.arange(0, n_dim, 2, dtype=jnp.float32) * (-math.log(max_n_times) / n_dim)
    )
    ang = position * div
    return jnp.stack([jnp.sin(ang), jnp.cos(ang)], axis=-1).reshape(n_times, n_dim)


def _sc_body(idx_hbm, emb_hbm, tt_hbm, out_hbm,
             idx_v, spat_v, tt_stage0, tt_stage1, buf0, buf1,
             sem_g, sem_t0, sem_t1, sem0, sem1,
             *, n_chans, n_times):
    core = lax.axis_index("c")
    sub = lax.axis_index("s")
    w = sub * 2 + core

    pltpu.sync_copy(idx_hbm.at[w], idx_v)
    gather = pltpu.make_async_copy(emb_hbm.at[idx_v], spat_v, sem_g)
    gather.start()

    n_chunks = n_times // CHUNK
    kspat = SPAT_DIM // LANES
    ktime = TIME_DIM // LANES
    bufs = [buf0, buf1]
    sems = [sem0, sem1]
    stages = [tt_stage0, tt_stage1]
    tsems = [sem_t0, sem_t1]

    def compose_and_fire(c, q, prev_cp, spat_regs):
        buf, sem = bufs[q], sems[q]
        dst = out_hbm.at[w, pl.ds(c * n_times + q * CHUNK, CHUNK), :]
        cp = pltpu.make_async_copy(buf, dst, sem)
        if prev_cp is not None:
            prev_cp.wait()

        def row_body(r, _, buf=buf, spat_regs=spat_regs):
            for k in range(kspat):
                buf[r, pl.ds(k * LANES, LANES)] = spat_regs[k]
            return 0

        lax.fori_loop(0, CHUNK, row_body, 0)
        cp.start()
        return cp

    prev = [None, None]
    slabs = CHUNK // TSTAGE
    n_stages = n_chunks * slabs
    scopies = []
    for t in range(n_stages):
        scopies.append(pltpu.make_async_copy(
            tt_hbm.at[pl.ds(t * TSTAGE, TSTAGE), :], stages[t & 1],
            tsems[t & 1]))
    scopies[0].start()
    for t in range(n_stages):
        if t + 1 < n_stages:
            scopies[t + 1].start()
        scopies[t].wait()
        q, s = divmod(t, slabs)

        def tt_body(r, _, buf=bufs[q], stage=stages[t & 1], s=s):
            for k in range(ktime):
                buf[s * TSTAGE + r, pl.ds(SPAT_DIM + k * LANES, LANES)] = (
                    stage[r, pl.ds(k * LANES, LANES)])
            return 0

        lax.fori_loop(0, TSTAGE, tt_body, 0)
    gather.wait()

    for c in range(n_chans):
        spat_regs = [spat_v[c, pl.ds(k * LANES, LANES)] for k in range(kspat)]
        for q in range(n_chunks):
            prev[q] = compose_and_fire(c, q, prev[q], spat_regs)
    prev[0].wait()
    prev[1].wait()


def kernel(local_features, ch_idxs, emb_table):
    B, n_chans_times, emb_dim = local_features.shape
    n_chans = ch_idxs.shape[1]
    n_times = n_chans_times // n_chans

    tt = _time_table(n_times, TIME_DIM, MAX_N_TIMES).astype(local_features.dtype)
    emb_wide = 256
    emb_padded = jnp.pad(emb_table, ((0, 0), (0, emb_wide - SPAT_DIM)))
    tt_padded = jnp.pad(tt, ((0, 0), (0, emb_wide - TIME_DIM)))
    idx_wide = 24
    idx_padded = jnp.pad(ch_idxs, ((0, 0), (0, idx_wide - n_chans)))

    body = functools.partial(_sc_body, n_chans=n_chans, n_times=n_times)
    run = pl.kernel(
        body,
        out_type=jax.ShapeDtypeStruct((B, n_chans_times, emb_dim),
                                      local_features.dtype),
        mesh=plsc.VectorSubcoreMesh(core_axis_name="c", subcore_axis_name="s"),
        scratch_types=[
            pltpu.VMEM((idx_wide,), jnp.int32),
            pltpu.VMEM((idx_wide, emb_wide), jnp.float32),
            pltpu.VMEM((TSTAGE, emb_wide), jnp.float32),
            pltpu.VMEM((TSTAGE, emb_wide), jnp.float32),
            pltpu.VMEM((CHUNK, emb_dim), jnp.float32),
            pltpu.VMEM((CHUNK, emb_dim), jnp.float32),
            pltpu.SemaphoreType.DMA,
            pltpu.SemaphoreType.DMA,
            pltpu.SemaphoreType.DMA,
            pltpu.SemaphoreType.DMA,
            pltpu.SemaphoreType.DMA,
        ],
    )
    return run(idx_padded, emb_padded, tt_padded)
